# baseline (speedup 1.0000x reference)
import functools

import jax
import jax.numpy as jnp
from jax import lax
from jax.experimental import pallas as pl
from jax.experimental.pallas import tpu as pltpu
from jax.experimental.pallas import tpu_sc as plsc

VOCAB = 1000000
EMBED = 64
T = 200

NC, NS = 2, 16
NW = NC * NS
LANES = 16


def _emb_kernel(n_rows: int):
    rows_per_w = n_rows // NW
    chunk = 800
    seqs = chunk // T
    n_chunks = rows_per_w // chunk

    mesh = plsc.VectorSubcoreMesh(core_axis_name="c", subcore_axis_name="s")

    @functools.partial(
        pl.kernel,
        out_type=jax.ShapeDtypeStruct((n_rows, EMBED), jnp.float32),
        mesh=mesh,
        scratch_types=[
            pltpu.VMEM((chunk,), jnp.int32),
            pltpu.VMEM((chunk, EMBED), jnp.float32),
            pltpu.VMEM((T, EMBED), jnp.float32),
            pltpu.SemaphoreType.DMA,
        ],
        compiler_params=pltpu.CompilerParams(use_tc_tiling_on_sc=False),
    )
    def body(idx_hbm, tbl_hbm, pos_hbm, out_hbm, idx_v, tok_v, pos_v, sem):
        wid = lax.axis_index("s") * NC + lax.axis_index("c")
        base = wid * rows_per_w
        pltpu.sync_copy(pos_hbm, pos_v)

        def add_pos(t, _):
            for j in range(EMBED // LANES):
                sl = pl.ds(j * LANES, LANES)
                pv = pos_v[t, sl]
                for s in range(seqs):
                    r = s * T + t
                    tok_v[r, sl] = tok_v[r, sl] + pv
            return 0

        for c in range(n_chunks):
            off = base + c * chunk
            pltpu.sync_copy(idx_hbm.at[pl.ds(off, chunk)], idx_v)
            pltpu.async_copy(tbl_hbm.at[idx_v], tok_v, sem).wait()
            lax.fori_loop(0, T, add_pos, 0)
            pltpu.sync_copy(tok_v, out_hbm.at[pl.ds(off, chunk)])

    return body


def kernel(idx, token_table, pos_table):
    B, Tv = idx.shape
    flat_idx = idx.reshape(B * Tv).astype(jnp.int32)
    out = _emb_kernel(B * Tv)(flat_idx, token_table, pos_table)
    return out.reshape(B, Tv, EMBED)


# speedup vs baseline: 1.0803x; 1.0803x over previous
import functools

import jax
import jax.numpy as jnp
from jax import lax
from jax.experimental import pallas as pl
from jax.experimental.pallas import tpu as pltpu
from jax.experimental.pallas import tpu_sc as plsc

VOCAB = 1000000
EMBED = 64
T = 200

NC, NS = 2, 16
NW = NC * NS
LANES = 16


def _emb_kernel(n_rows: int):
    rows_per_w = n_rows // NW
    chunk = 800
    seqs = chunk // T
    n_chunks = rows_per_w // chunk

    mesh = plsc.VectorSubcoreMesh(core_axis_name="c", subcore_axis_name="s")

    @functools.partial(
        pl.kernel,
        out_type=jax.ShapeDtypeStruct((n_rows, EMBED), jnp.float32),
        mesh=mesh,
        scratch_types=[
            pltpu.VMEM((chunk,), jnp.int32),
            pltpu.VMEM((chunk,), jnp.int32),
            pltpu.VMEM((chunk, EMBED), jnp.float32),
            pltpu.VMEM((chunk, EMBED), jnp.float32),
            pltpu.VMEM((T, EMBED), jnp.float32),
            pltpu.SemaphoreType.DMA,
            pltpu.SemaphoreType.DMA,
            pltpu.SemaphoreType.DMA,
            pltpu.SemaphoreType.DMA,
        ],
        compiler_params=pltpu.CompilerParams(use_tc_tiling_on_sc=False),
    )
    def body(idx_hbm, tbl_hbm, pos_hbm, out_hbm,
             idx0, idx1, tok0, tok1, pos_v, g0, g1, o0, o1):
        wid = lax.axis_index("s") * NC + lax.axis_index("c")
        base = wid * rows_per_w
        idxs = (idx0, idx1)
        toks = (tok0, tok1)
        gsems = (g0, g1)
        osems = (o0, o1)
        pltpu.sync_copy(pos_hbm, pos_v)

        def add_pos(tok_v):
            def step(t, _):
                for j in range(EMBED // LANES):
                    sl = pl.ds(j * LANES, LANES)
                    pv = pos_v[t, sl]
                    for s in range(seqs):
                        r = s * T + t
                        tok_v[r, sl] = tok_v[r, sl] + pv
                return 0
            lax.fori_loop(0, T, step, 0)

        pltpu.sync_copy(idx_hbm.at[pl.ds(base, chunk)], idxs[0])
        pltpu.make_async_copy(tbl_hbm.at[idxs[0]], toks[0], gsems[0]).start()

        for c in range(n_chunks):
            cur, nxt = c % 2, (c + 1) % 2
            if c + 1 < n_chunks:
                off_n = base + (c + 1) * chunk
                pltpu.sync_copy(idx_hbm.at[pl.ds(off_n, chunk)], idxs[nxt])
                if c >= 1:
                    pltpu.make_async_copy(toks[nxt], out_hbm.at[pl.ds(0, chunk)],
                                          osems[nxt]).wait()
                pltpu.make_async_copy(tbl_hbm.at[idxs[nxt]], toks[nxt],
                                      gsems[nxt]).start()
            pltpu.make_async_copy(tbl_hbm.at[idxs[cur]], toks[cur],
                                  gsems[cur]).wait()
            add_pos(toks[cur])
            off = base + c * chunk
            pltpu.make_async_copy(toks[cur], out_hbm.at[pl.ds(off, chunk)],
                                  osems[cur]).start()

        for cur in (n_chunks % 2, (n_chunks + 1) % 2):
            pltpu.make_async_copy(toks[cur], out_hbm.at[pl.ds(0, chunk)],
                                  osems[cur]).wait()

    return body


def kernel(idx, token_table, pos_table):
    B, Tv = idx.shape
    flat_idx = idx.reshape(B * Tv).astype(jnp.int32)
    out = _emb_kernel(B * Tv)(flat_idx, token_table, pos_table)
    return out.reshape(B, Tv, EMBED)


# speedup vs baseline: 1.0818x; 1.0014x over previous
import functools

import jax
import jax.numpy as jnp
from jax import lax
from jax.experimental import pallas as pl
from jax.experimental.pallas import tpu as pltpu
from jax.experimental.pallas import tpu_sc as plsc

VOCAB = 1000000
EMBED = 64

NC, NS = 2, 16
NW = NC * NS
LANES = 16


def _emb_kernel(B: int, T: int):
    rows_per_w = B // NW
    seqs = 4
    chunk = seqs * T
    n_chunks = rows_per_w // seqs

    mesh = plsc.VectorSubcoreMesh(core_axis_name="c", subcore_axis_name="s")

    @functools.partial(
        pl.kernel,
        out_type=jax.ShapeDtypeStruct((B, T, EMBED), jnp.float32),
        mesh=mesh,
        scratch_types=[
            pltpu.VMEM((seqs, T), jnp.int32),
            pltpu.VMEM((seqs, T), jnp.int32),
            pltpu.VMEM((seqs, T, EMBED), jnp.float32),
            pltpu.VMEM((seqs, T, EMBED), jnp.float32),
            pltpu.VMEM((T, EMBED), jnp.float32),
            pltpu.SemaphoreType.DMA,
            pltpu.SemaphoreType.DMA,
            pltpu.SemaphoreType.DMA,
            pltpu.SemaphoreType.DMA,
        ],
        compiler_params=pltpu.CompilerParams(use_tc_tiling_on_sc=False),
    )
    def body(idx_hbm, tbl_hbm, pos_hbm, out_hbm,
             idx0, idx1, tok0, tok1, pos_v, g0, g1, o0, o1):
        wid = lax.axis_index("s") * NC + lax.axis_index("c")
        base = wid * rows_per_w
        idxs = (idx0, idx1)
        toks = (tok0, tok1)
        gsems = (g0, g1)
        osems = (o0, o1)
        pltpu.sync_copy(pos_hbm, pos_v)

        def add_pos(tok_v):
            def step(t, _):
                for j in range(EMBED // LANES):
                    sl = pl.ds(j * LANES, LANES)
                    pv = pos_v[t, sl]
                    for s in range(seqs):
                        tok_v[s, t, sl] = tok_v[s, t, sl] + pv
                return 0
            lax.fori_loop(0, T, step, 0)

        def start_gathers(buf):
            for s in range(seqs):
                pltpu.make_async_copy(tbl_hbm.at[idxs[buf].at[s]],
                                      toks[buf].at[s], gsems[buf]).start()

        def wait_gathers(buf):
            for s in range(seqs):
                pltpu.make_async_copy(tbl_hbm.at[idxs[buf].at[s]],
                                      toks[buf].at[s], gsems[buf]).wait()

        pltpu.sync_copy(idx_hbm.at[pl.ds(base, seqs)], idxs[0])
        start_gathers(0)

        for c in range(n_chunks):
            cur, nxt = c % 2, (c + 1) % 2
            if c + 1 < n_chunks:
                off_n = base + (c + 1) * seqs
                pltpu.sync_copy(idx_hbm.at[pl.ds(off_n, seqs)], idxs[nxt])
                if c >= 1:
                    pltpu.make_async_copy(toks[nxt], out_hbm.at[pl.ds(0, seqs)],
                                          osems[nxt]).wait()
                start_gathers(nxt)
            wait_gathers(cur)
            add_pos(toks[cur])
            off = base + c * seqs
            pltpu.make_async_copy(toks[cur], out_hbm.at[pl.ds(off, seqs)],
                                  osems[cur]).start()

        for cur in (n_chunks % 2, (n_chunks + 1) % 2):
            pltpu.make_async_copy(toks[cur], out_hbm.at[pl.ds(0, seqs)],
                                  osems[cur]).wait()

    return body


def kernel(idx, token_table, pos_table):
    B, Tv = idx.shape
    return _emb_kernel(B, Tv)(idx.astype(jnp.int32), token_table, pos_table)
